# Initial kernel scaffold; baseline (speedup 1.0000x reference)
import jax
import jax.numpy as jnp
from jax.experimental import pallas as pl


def kernel(x, edge_index, batch, W0, b0, W1, b1, W2, b2, Wout, bout):
    raise NotImplementedError("write your pallas kernel here")


# speedup vs baseline: 11.3873x; 11.3873x over previous
import functools

import jax
import jax.numpy as jnp
from jax import lax
from jax.experimental import pallas as pl
from jax.experimental.pallas import tpu as pltpu
from jax.experimental.pallas import tpu_sc as plsc

_N, _E, _D, _H, _B = 10000, 320000, 128, 128, 64
_NP = 10240
_NC, _NS = 2, 16
_NW = _NC * _NS
_EPW = _E // _NW
_K = 80
_NCH = _EPW // _K
_RPT = _NP // _NS

_f32 = jnp.float32
_i32 = jnp.int32

_MESH = plsc.VectorSubcoreMesh(core_axis_name="c", subcore_axis_name="s")


@functools.partial(
    pl.kernel,
    out_type=jax.ShapeDtypeStruct((_NC, _NP), _f32),
    mesh=_MESH,
    scratch_types=[
        pltpu.VMEM((_K,), _i32),
        pltpu.VMEM((_K,), _f32),
        pltpu.VMEM((_RPT,), _f32),
        pltpu.VMEM_SHARED((_NP,), _f32),
    ],
)
def _deg_kernel(dst_hbm, out_hbm, idx_v, ones_v, zero_v, deg_sh):
    c = lax.axis_index("c")
    s = lax.axis_index("s")
    base = (c * _NS + s) * _EPW

    zeros16 = jnp.zeros((16,), _f32)
    ones16 = jnp.ones((16,), _f32)
    for j in range(_K // 16):
        ones_v[pl.ds(j * 16, 16)] = ones16

    @pl.loop(0, _RPT // 16)
    def _(i):
        zero_v[pl.ds(i * 16, 16)] = zeros16

    pltpu.sync_copy(zero_v, deg_sh.at[pl.ds(s * _RPT, _RPT)])
    plsc.subcore_barrier()

    @pl.loop(0, _NCH)
    def _(i):
        pltpu.sync_copy(dst_hbm.at[pl.ds(base + i * _K, _K)], idx_v)
        pltpu.sync_copy(ones_v, deg_sh.at[idx_v], add=True)

    plsc.subcore_barrier()
    pltpu.sync_copy(deg_sh.at[pl.ds(s * _RPT, _RPT)],
                    out_hbm.at[c, pl.ds(s * _RPT, _RPT)])


@functools.partial(
    pl.kernel,
    out_type=jax.ShapeDtypeStruct((_NC, _NP, _H), _f32),
    mesh=_MESH,
    scratch_types=[
        pltpu.VMEM((_K,), _i32),
        pltpu.VMEM((_K,), _i32),
        pltpu.VMEM((_K, _H), _f32),
        pltpu.VMEM_SHARED((_NP, _H), _f32),
        pltpu.SemaphoreType.DMA,
    ],
)
def _prop_kernel(g_hbm, src_hbm, dst_hbm, out_hbm,
                 sidx_v, didx_v, rows_v, acc_sh, sem):
    c = lax.axis_index("c")
    s = lax.axis_index("s")
    base = (c * _NS + s) * _EPW

    zeros16 = jnp.zeros((16,), _f32)

    @pl.loop(0, _K)
    def _(i):
        for cc in range(_H // 16):
            rows_v[i, pl.ds(cc * 16, 16)] = zeros16

    for r in range(_RPT // _K):
        pltpu.sync_copy(rows_v, acc_sh.at[pl.ds(s * _RPT + r * _K, _K)])
    plsc.subcore_barrier()

    @pl.loop(0, _NCH)
    def _(i):
        off = base + i * _K
        pltpu.sync_copy(src_hbm.at[pl.ds(off, _K)], sidx_v)
        pltpu.sync_copy(dst_hbm.at[pl.ds(off, _K)], didx_v)
        pltpu.async_copy(g_hbm.at[sidx_v], rows_v, sem).wait()
        pltpu.sync_copy(rows_v, acc_sh.at[didx_v], add=True)

    plsc.subcore_barrier()
    pltpu.sync_copy(acc_sh.at[pl.ds(s * _RPT, _RPT)],
                    out_hbm.at[c, pl.ds(s * _RPT, _RPT)])


def _mm0_body(x_ref, w_ref, degp_ref, t_ref, g_ref, dinv_ref, invdeg_ref):
    deg = degp_ref[:, 0:1] + degp_ref[:, 1:2] + 1.0
    dinv = lax.rsqrt(deg)
    invdeg = 1.0 / deg
    t = jnp.dot(x_ref[...], w_ref[...], preferred_element_type=_f32)
    t_ref[...] = t
    g_ref[...] = t * dinv
    dinv_ref[...] = dinv
    invdeg_ref[...] = invdeg


def _mm0(x, w, degp):
    blk = 1024
    grid = _NP // blk
    return pl.pallas_call(
        _mm0_body,
        grid=(grid,),
        in_specs=[
            pl.BlockSpec((blk, _D), lambda i: (i, 0)),
            pl.BlockSpec((_D, _H), lambda i: (0, 0)),
            pl.BlockSpec((blk, _NC), lambda i: (i, 0)),
        ],
        out_specs=[
            pl.BlockSpec((blk, _H), lambda i: (i, 0)),
            pl.BlockSpec((blk, _H), lambda i: (i, 0)),
            pl.BlockSpec((blk, 1), lambda i: (i, 0)),
            pl.BlockSpec((blk, 1), lambda i: (i, 0)),
        ],
        out_shape=[
            jax.ShapeDtypeStruct((_NP, _H), _f32),
            jax.ShapeDtypeStruct((_NP, _H), _f32),
            jax.ShapeDtypeStruct((_NP, 1), _f32),
            jax.ShapeDtypeStruct((_NP, 1), _f32),
        ],
    )(x, w, degp)


def _comb_body(acc_ref, t_ref, dinv_ref, invdeg_ref, b_ref, w_ref,
               t2_ref, g2_ref):
    dinv = dinv_ref[...]
    pre = (dinv * (acc_ref[0] + acc_ref[1])
           + invdeg_ref[...] * t_ref[...] + b_ref[...])
    h = jnp.maximum(pre, 0.0)
    t2 = jnp.dot(h, w_ref[...], preferred_element_type=_f32)
    t2_ref[...] = t2
    g2_ref[...] = t2 * dinv


def _comb(acc, t, dinv, invdeg, b, w):
    blk = 1024
    grid = _NP // blk
    return pl.pallas_call(
        _comb_body,
        grid=(grid,),
        in_specs=[
            pl.BlockSpec((_NC, blk, _H), lambda i: (0, i, 0)),
            pl.BlockSpec((blk, _H), lambda i: (i, 0)),
            pl.BlockSpec((blk, 1), lambda i: (i, 0)),
            pl.BlockSpec((blk, 1), lambda i: (i, 0)),
            pl.BlockSpec((1, _H), lambda i: (0, 0)),
            pl.BlockSpec((_H, _H), lambda i: (0, 0)),
        ],
        out_specs=[
            pl.BlockSpec((blk, _H), lambda i: (i, 0)),
            pl.BlockSpec((blk, _H), lambda i: (i, 0)),
        ],
        out_shape=[
            jax.ShapeDtypeStruct((_NP, _H), _f32),
            jax.ShapeDtypeStruct((_NP, _H), _f32),
        ],
    )(acc, t, dinv, invdeg, b, w)


def _final_body(acc_ref, t_ref, dinv_ref, invdeg_ref, b_ref, batch_ref,
                wout_ref, bout_ref, out_ref, pool_ref, cnt_ref):
    i = pl.program_id(0)

    @pl.when(i == 0)
    def _():
        pool_ref[...] = jnp.zeros_like(pool_ref)
        cnt_ref[...] = jnp.zeros_like(cnt_ref)

    h3 = (dinv_ref[...] * (acc_ref[0] + acc_ref[1])
          + invdeg_ref[...] * t_ref[...] + b_ref[...])
    seg = lax.broadcasted_iota(_i32, (_B, batch_ref.shape[1]), 0)
    m = (seg == batch_ref[...]).astype(_f32)
    pool_ref[...] += jnp.dot(m, h3, preferred_element_type=_f32)
    cnt_ref[...] += jnp.sum(m, axis=1, keepdims=True)

    @pl.when(i == pl.num_programs(0) - 1)
    def _():
        pooled = pool_ref[...] / jnp.maximum(cnt_ref[...], 1.0)
        out_ref[...] = (jnp.dot(pooled, wout_ref[...],
                                preferred_element_type=_f32) + bout_ref[...])


def _final(acc, t, dinv, invdeg, b, batch2d, wout, bout):
    blk = 1024
    grid = _NP // blk
    return pl.pallas_call(
        _final_body,
        grid=(grid,),
        in_specs=[
            pl.BlockSpec((_NC, blk, _H), lambda i: (0, i, 0)),
            pl.BlockSpec((blk, _H), lambda i: (i, 0)),
            pl.BlockSpec((blk, 1), lambda i: (i, 0)),
            pl.BlockSpec((blk, 1), lambda i: (i, 0)),
            pl.BlockSpec((1, _H), lambda i: (0, 0)),
            pl.BlockSpec((1, blk), lambda i: (0, i)),
            pl.BlockSpec((_H, _H), lambda i: (0, 0)),
            pl.BlockSpec((1, _H), lambda i: (0, 0)),
        ],
        out_specs=pl.BlockSpec((_B, _H), lambda i: (0, 0)),
        out_shape=jax.ShapeDtypeStruct((_B, _H), _f32),
        scratch_shapes=[
            pltpu.VMEM((_B, _H), _f32),
            pltpu.VMEM((_B, 1), _f32),
        ],
    )(acc, t, dinv, invdeg, b, batch2d, wout, bout)


@jax.jit
def kernel(x, edge_index, batch, W0, b0, W1, b1, W2, b2, Wout, bout):
    src = edge_index[0]
    dst = edge_index[1]
    xp = jnp.pad(x, ((0, _NP - _N), (0, 0)))
    batchp = jnp.pad(batch, (0, _NP - _N), constant_values=_B)[None, :]

    degp = _deg_kernel(dst)
    t0, g0, dinv, invdeg = _mm0(xp, W0, degp.T)

    acc0 = _prop_kernel(g0, src, dst)
    t1, g1 = _comb(acc0, t0, dinv, invdeg, b0[None, :], W1)

    acc1 = _prop_kernel(g1, src, dst)
    t2, g2 = _comb(acc1, t1, dinv, invdeg, b1[None, :], W2)

    acc2 = _prop_kernel(g2, src, dst)
    out = _final(acc2, t2, dinv, invdeg, b2[None, :], batchp, Wout,
                 bout[None, :])
    return out
